# baseline (speedup 1.0000x reference)
import functools

import numpy as np
import jax
import jax.numpy as jnp
from jax.experimental import pallas as pl
from jax.experimental.pallas import tpu as pltpu

_BN_EPS = 1e-5


def _stats_kernel(x_ref, o_ref):
    j = pl.program_id(1)

    @pl.when(j == 0)
    def _init():
        o_ref[...] = jnp.zeros_like(o_ref)

    x = x_ref[...]
    o_ref[0, 0:1, :] += jnp.sum(x, axis=0, keepdims=True)
    o_ref[0, 1:2, :] += jnp.sum(x * x, axis=0, keepdims=True)


def _forward_kernel(x_ref, part_ref, gb_ref, wint_ref, w1_ref, w2_ref, b_ref,
                    o_ref, *, batch, ncores, groups, tg, chans, filters,
                    hidden, classes):
    s = part_ref[0:1, :]
    q = part_ref[1:2, :]
    for c in range(1, ncores):
        s = s + part_ref[2 * c:2 * c + 1, :]
        q = q + part_ref[2 * c + 1:2 * c + 2, :]
    inv_b = 1.0 / batch
    mean = s * inv_b
    var = q * inv_b - mean * mean
    scale = gb_ref[0:1, :] * jax.lax.rsqrt(var + _BN_EPS)
    shift = gb_ref[1:2, :] - mean * scale

    x = x_ref[...] * scale + shift

    win = (tg + 2) * chans
    half = tg * filters // 2
    pooled_parts = []
    for g in range(groups):
        xw = x[:, g * tg * chans: g * tg * chans + win]
        cg = jnp.dot(xw, wint_ref[...], preferred_element_type=jnp.float32)
        cg = jnp.maximum(cg + b_ref[0:1, :], 0.0)
        pooled_parts.append(jnp.maximum(cg[:, :half], cg[:, half:]))
    pooled = jnp.concatenate(pooled_parts, axis=1)

    h = jnp.dot(pooled, w1_ref[...], preferred_element_type=jnp.float32)
    h = jnp.maximum(h + b_ref[1:2, :hidden], 0.0)
    logits = jnp.dot(h, w2_ref[...], preferred_element_type=jnp.float32)
    logits = logits + b_ref[2:3, :classes]
    o_ref[...] = jax.nn.sigmoid(logits).astype(o_ref.dtype)


def _build_group_weight(w_toep, C, L, F, K, tg, pooling):
    pool_len = L // pooling
    w3 = w_toep.reshape(C, L, pooling * pool_len * F)[:, :K,
                                                      pool_len * F:pool_len * F + F]
    rows, cols = [], []
    for u in range(tg // pooling):
        for p in range(pooling):
            t_local = pooling * u + p
            for k in range(K):
                r0 = (t_local + k) * C
                c0 = (p * (tg // pooling) + u) * F
                rr, cc = np.meshgrid(np.arange(r0, r0 + C),
                                     np.arange(c0, c0 + F), indexing="ij")
                rows.append(rr.ravel())
                cols.append(cc.ravel())
    flat = (np.concatenate(rows) * (tg * F) + np.concatenate(cols)).astype(np.int32)
    vals = jnp.broadcast_to(
        w3.transpose(1, 0, 2)[None, None],
        (tg // pooling, pooling, K, C, F)).reshape(-1)
    wint = jnp.zeros(((tg + 2) * C) * (tg * F), jnp.float32)
    wint = wint.at[jnp.asarray(flat)].set(vals)
    return wint.reshape((tg + 2) * C, tg * F)


def kernel(x, w_toep, fc1_w, fc2_w, bias_pack, gb_pack):
    B, C, L = x.shape
    H = fc1_w.shape[1]
    classes = fc2_w.shape[1]
    n_conv = w_toep.shape[1]
    pooling = 2
    pool_len = L // pooling
    F = n_conv // (pooling * pool_len)
    K = 3
    TG = 16
    G = L // TG
    LCP = (L + 2) * C

    x_t = jnp.pad(x.transpose(0, 2, 1), ((0, 0), (1, 1), (0, 0))).reshape(B, LCP)
    gb_t = jnp.pad(gb_pack.reshape(2, C, L).transpose(0, 2, 1).reshape(2, C * L),
                   ((0, 0), (C, C)))
    wint = _build_group_weight(w_toep, C, L, F, K, TG, pooling)
    bpk = jnp.zeros((3, TG * F), jnp.float32)
    bpk = bpk.at[0, :].set(jnp.tile(bias_pack[0, :F], TG))
    bpk = bpk.at[1, :H].set(bias_pack[1, :H])
    bpk = bpk.at[2, :classes].set(bias_pack[2, :classes])

    TB = 128 if B % 128 == 0 else B
    n_tiles = B // TB
    ncores = 2 if n_tiles % 2 == 0 else 1
    half_tiles = n_tiles // ncores
    vmem_limit = 32 * 1024 * 1024

    parts = pl.pallas_call(
        _stats_kernel,
        out_shape=jax.ShapeDtypeStruct((ncores, 2, LCP), jnp.float32),
        grid_spec=pltpu.PrefetchScalarGridSpec(
            num_scalar_prefetch=0,
            grid=(ncores, half_tiles),
            in_specs=[pl.BlockSpec((TB, LCP),
                                   lambda i, j: (i * half_tiles + j, 0))],
            out_specs=pl.BlockSpec((1, 2, LCP), lambda i, j: (i, 0, 0))),
        compiler_params=pltpu.CompilerParams(
            dimension_semantics=("parallel", "arbitrary"),
            vmem_limit_bytes=vmem_limit),
    )(x_t)
    parts = parts.reshape(2 * ncores, LCP)

    fwd = functools.partial(_forward_kernel, batch=B, ncores=ncores, groups=G,
                            tg=TG, chans=C, filters=F, hidden=H,
                            classes=classes)
    out = pl.pallas_call(
        fwd,
        out_shape=jax.ShapeDtypeStruct((B, classes), jnp.float32),
        grid_spec=pltpu.PrefetchScalarGridSpec(
            num_scalar_prefetch=0,
            grid=(n_tiles,),
            in_specs=[
                pl.BlockSpec((TB, LCP), lambda i: (i, 0)),
                pl.BlockSpec((2 * ncores, LCP), lambda i: (0, 0)),
                pl.BlockSpec((2, LCP), lambda i: (0, 0)),
                pl.BlockSpec(((TG + 2) * C, TG * F), lambda i: (0, 0)),
                pl.BlockSpec((pool_len * F, H), lambda i: (0, 0)),
                pl.BlockSpec((H, classes), lambda i: (0, 0)),
                pl.BlockSpec((3, TG * F), lambda i: (0, 0)),
            ],
            out_specs=pl.BlockSpec((TB, classes), lambda i: (i, 0))),
        compiler_params=pltpu.CompilerParams(
            dimension_semantics=("parallel",),
            vmem_limit_bytes=vmem_limit),
    )(x_t, parts, gb_t, wint, fc1_w, fc2_w, bpk)
    return out


# speedup vs baseline: 1.3790x; 1.3790x over previous
import functools

import numpy as np
import jax
import jax.numpy as jnp
from jax.experimental import pallas as pl
from jax.experimental.pallas import tpu as pltpu

_BN_EPS = 1e-5


def _stats_kernel(x_ref, o_ref):
    j = pl.program_id(1)

    @pl.when(j == 0)
    def _init():
        o_ref[...] = jnp.zeros_like(o_ref)

    x = x_ref[...]
    o_ref[0, 0:1, :] += jnp.sum(x, axis=0, keepdims=True)
    o_ref[0, 1:2, :] += jnp.sum(x * x, axis=0, keepdims=True)


def _forward_kernel(x_ref, part_ref, gb_ref, wint_ref, w1_ref, w2_ref, b_ref,
                    o_ref, *, batch, ncores, groups, tg, chans, filters,
                    hidden, classes):
    s = part_ref[0:1, :]
    q = part_ref[1:2, :]
    for c in range(1, ncores):
        s = s + part_ref[2 * c:2 * c + 1, :]
        q = q + part_ref[2 * c + 1:2 * c + 2, :]
    inv_b = 1.0 / batch
    mean = s * inv_b
    var = q * inv_b - mean * mean
    scale = gb_ref[0:1, :] * jax.lax.rsqrt(var + _BN_EPS)
    shift = gb_ref[1:2, :] - mean * scale

    x = x_ref[...] * scale + shift

    win = (tg + 2) * chans
    half = tg * filters // 2
    pooled_parts = []
    for g in range(groups):
        xw = x[:, g * tg * chans: g * tg * chans + win]
        cg = jnp.dot(xw, wint_ref[...], preferred_element_type=jnp.float32)
        cg = jnp.maximum(cg + b_ref[0:1, :], 0.0)
        pooled_parts.append(jnp.maximum(cg[:, :half], cg[:, half:]))
    pooled = jnp.concatenate(pooled_parts, axis=1)

    h = jnp.dot(pooled, w1_ref[...], preferred_element_type=jnp.float32)
    h = jnp.maximum(h + b_ref[1:2, :hidden], 0.0)
    logits = jnp.dot(h, w2_ref[...], preferred_element_type=jnp.float32)
    logits = logits + b_ref[2:3, :classes]
    o_ref[...] = jax.nn.sigmoid(logits).astype(o_ref.dtype)


def _build_group_weight(w_toep, C, L, F, K, tg, pooling):
    pool_len = L // pooling
    w3 = w_toep.reshape(C, L, pooling * pool_len * F)[:, :K,
                                                      pool_len * F:pool_len * F + F]
    wint = jnp.zeros(((tg + 2) * C, tg * F), jnp.float32)
    for k in range(K):
        a_k = np.zeros((tg + 2, tg), np.float32)
        for u in range(tg // pooling):
            for p in range(pooling):
                t_local = pooling * u + p
                j = p * (tg // pooling) + u
                a_k[t_local + k, j] = 1.0
        wint = wint + jnp.kron(jnp.asarray(a_k), w3[:, k, :])
    return wint


def kernel(x, w_toep, fc1_w, fc2_w, bias_pack, gb_pack):
    B, C, L = x.shape
    H = fc1_w.shape[1]
    classes = fc2_w.shape[1]
    n_conv = w_toep.shape[1]
    pooling = 2
    pool_len = L // pooling
    F = n_conv // (pooling * pool_len)
    K = 3
    TG = 16
    G = L // TG
    LCP = (L + 2) * C

    x_t = jnp.pad(x.transpose(0, 2, 1), ((0, 0), (1, 1), (0, 0))).reshape(B, LCP)
    gb_t = jnp.pad(gb_pack.reshape(2, C, L).transpose(0, 2, 1).reshape(2, C * L),
                   ((0, 0), (C, C)))
    wint = _build_group_weight(w_toep, C, L, F, K, TG, pooling)
    bpk = jnp.zeros((3, TG * F), jnp.float32)
    bpk = bpk.at[0, :].set(jnp.tile(bias_pack[0, :F], TG))
    bpk = bpk.at[1, :H].set(bias_pack[1, :H])
    bpk = bpk.at[2, :classes].set(bias_pack[2, :classes])

    TB = 128 if B % 128 == 0 else B
    n_tiles = B // TB
    ncores = 2 if n_tiles % 2 == 0 else 1
    half_tiles = n_tiles // ncores
    vmem_limit = 32 * 1024 * 1024

    parts = pl.pallas_call(
        _stats_kernel,
        out_shape=jax.ShapeDtypeStruct((ncores, 2, LCP), jnp.float32),
        grid_spec=pltpu.PrefetchScalarGridSpec(
            num_scalar_prefetch=0,
            grid=(ncores, half_tiles),
            in_specs=[pl.BlockSpec((TB, LCP),
                                   lambda i, j: (i * half_tiles + j, 0))],
            out_specs=pl.BlockSpec((1, 2, LCP), lambda i, j: (i, 0, 0))),
        compiler_params=pltpu.CompilerParams(
            dimension_semantics=("parallel", "arbitrary"),
            vmem_limit_bytes=vmem_limit),
    )(x_t)
    parts = parts.reshape(2 * ncores, LCP)

    fwd = functools.partial(_forward_kernel, batch=B, ncores=ncores, groups=G,
                            tg=TG, chans=C, filters=F, hidden=H,
                            classes=classes)
    out = pl.pallas_call(
        fwd,
        out_shape=jax.ShapeDtypeStruct((B, classes), jnp.float32),
        grid_spec=pltpu.PrefetchScalarGridSpec(
            num_scalar_prefetch=0,
            grid=(n_tiles,),
            in_specs=[
                pl.BlockSpec((TB, LCP), lambda i: (i, 0)),
                pl.BlockSpec((2 * ncores, LCP), lambda i: (0, 0)),
                pl.BlockSpec((2, LCP), lambda i: (0, 0)),
                pl.BlockSpec(((TG + 2) * C, TG * F), lambda i: (0, 0)),
                pl.BlockSpec((pool_len * F, H), lambda i: (0, 0)),
                pl.BlockSpec((H, classes), lambda i: (0, 0)),
                pl.BlockSpec((3, TG * F), lambda i: (0, 0)),
            ],
            out_specs=pl.BlockSpec((TB, classes), lambda i: (i, 0))),
        compiler_params=pltpu.CompilerParams(
            dimension_semantics=("parallel",),
            vmem_limit_bytes=vmem_limit),
    )(x_t, parts, gb_t, wint, fc1_w, fc2_w, bpk)
    return out


# speedup vs baseline: 1.7154x; 1.2439x over previous
import functools

import numpy as np
import jax
import jax.numpy as jnp
from jax.experimental import pallas as pl
from jax.experimental.pallas import tpu as pltpu

_BN_EPS = 1e-5


def _stats_kernel(x_ref, o_ref):
    j = pl.program_id(1)

    @pl.when(j == 0)
    def _init():
        o_ref[...] = jnp.zeros_like(o_ref)

    x = x_ref[...]
    o_ref[0, 0:1, :] += jnp.sum(x, axis=0, keepdims=True)
    o_ref[0, 1:2, :] += jnp.sum(x * x, axis=0, keepdims=True)


def _forward_kernel(x_ref, aff_ref, wc_ref, w1_ref, w2_ref, b_ref, o_ref, *,
                    groups, blk, chans, length, hidden, classes):
    xa = x_ref[...] * aff_ref[0:1, :] + aff_ref[1:2, :]
    xt = jnp.transpose(xa)
    tb = xt.shape[1]

    hb = blk // 2
    x3 = xt.reshape(chans, length, tb)
    x3 = jnp.pad(x3, ((0, 0), (hb, hb), (0, 0)))
    x_sb = jnp.stack([x3[:, m * blk:(m + 1) * blk, :]
                      for m in range(groups + 1)])
    x_sb = x_sb.reshape((groups + 1) * chans * blk, tb)

    pooled_parts = []
    for g in range(groups):
        xw = x_sb[g * chans * blk:(g + 2) * chans * blk, :]
        cg = jnp.dot(wc_ref[...], xw, preferred_element_type=jnp.float32)
        cg = jnp.maximum(cg + b_ref[:, 0:1], 0.0)
        half = cg.shape[0] // 2
        pooled_parts.append(jnp.maximum(cg[:half], cg[half:]))
    pooled = jnp.concatenate(pooled_parts, axis=0)

    h = jnp.dot(w1_ref[...], pooled, preferred_element_type=jnp.float32)
    h = jnp.maximum(h + b_ref[:hidden, 1:2], 0.0)
    logits = jnp.dot(w2_ref[...], h, preferred_element_type=jnp.float32)
    logits = logits + b_ref[:classes, 2:3]
    sig = jax.nn.sigmoid(logits)
    o_ref[...] = jnp.transpose(sig).astype(o_ref.dtype)


def _build_group_weight(w_toep, C, L, F, K, tg, pooling, blk):
    pool_len = L // pooling
    w3 = w_toep.reshape(C, L, pooling * pool_len * F)[:, :K,
                                                      pool_len * F:pool_len * F + F]
    P = np.zeros((K, 2 * blk, tg), np.float32)
    for u in range(tg // pooling):
        for p in range(pooling):
            t_local = pooling * u + p
            j = p * (tg // pooling) + u
            for k in range(K):
                P[k, t_local + k - 1 + blk // 2, j] = 1.0
    w = jnp.einsum('kDj,ckf->jfcD', jnp.asarray(P), w3)
    w = w.reshape(tg * F, C, 2, blk).transpose(0, 2, 1, 3)
    return w.reshape(tg * F, 2 * C * blk)


def kernel(x, w_toep, fc1_w, fc2_w, bias_pack, gb_pack):
    B, C, L = x.shape
    H = fc1_w.shape[1]
    classes = fc2_w.shape[1]
    n_conv = w_toep.shape[1]
    pooling = 2
    pool_len = L // pooling
    F = n_conv // (pooling * pool_len)
    K = 3
    TG = 16
    G = L // TG
    BLK = 16
    LC = C * L

    x2 = x.reshape(B, LC)
    wcT = _build_group_weight(w_toep, C, L, F, K, TG, pooling, BLK)
    bpk = jnp.zeros((TG * F, 3), jnp.float32)
    bpk = bpk.at[:, 0].set(jnp.tile(bias_pack[0, :F], TG))
    bpk = bpk.at[:H, 1].set(bias_pack[1, :H])
    bpk = bpk.at[:classes, 2].set(bias_pack[2, :classes])

    TBS = 128 if B % 128 == 0 else B
    ns_tiles = B // TBS
    ncores = 2 if ns_tiles % 2 == 0 else 1
    half_tiles = ns_tiles // ncores
    vmem_limit = 64 * 1024 * 1024

    parts = pl.pallas_call(
        _stats_kernel,
        out_shape=jax.ShapeDtypeStruct((ncores, 2, LC), jnp.float32),
        grid_spec=pltpu.PrefetchScalarGridSpec(
            num_scalar_prefetch=0,
            grid=(ncores, half_tiles),
            in_specs=[pl.BlockSpec((TBS, LC),
                                   lambda i, j: (i * half_tiles + j, 0))],
            out_specs=pl.BlockSpec((1, 2, LC), lambda i, j: (i, 0, 0))),
        compiler_params=pltpu.CompilerParams(
            dimension_semantics=("parallel", "arbitrary"),
            vmem_limit_bytes=vmem_limit),
    )(x2)

    s = parts[:, 0, :].sum(axis=0)
    q = parts[:, 1, :].sum(axis=0)
    mean = s / B
    var = q / B - mean * mean
    scale = gb_pack[0] * jax.lax.rsqrt(var + _BN_EPS)
    shift = gb_pack[1] - mean * scale
    aff = jnp.stack([scale, shift])

    TB = 256 if B % 256 == 0 else B
    n_tiles = B // TB
    fwd = functools.partial(_forward_kernel, groups=G, blk=BLK, chans=C,
                            length=L, hidden=H, classes=classes)
    out = pl.pallas_call(
        fwd,
        out_shape=jax.ShapeDtypeStruct((B, classes), jnp.float32),
        grid_spec=pltpu.PrefetchScalarGridSpec(
            num_scalar_prefetch=0,
            grid=(n_tiles,),
            in_specs=[
                pl.BlockSpec((TB, LC), lambda i: (i, 0)),
                pl.BlockSpec((2, LC), lambda i: (0, 0)),
                pl.BlockSpec((TG * F, 2 * C * BLK), lambda i: (0, 0)),
                pl.BlockSpec((H, pool_len * F), lambda i: (0, 0)),
                pl.BlockSpec((classes, H), lambda i: (0, 0)),
                pl.BlockSpec((TG * F, 3), lambda i: (0, 0)),
            ],
            out_specs=pl.BlockSpec((TB, classes), lambda i: (i, 0))),
        compiler_params=pltpu.CompilerParams(
            dimension_semantics=("parallel",),
            vmem_limit_bytes=vmem_limit),
    )(x2, aff, wcT, fc1_w.T, fc2_w.T, bpk)
    return out


# speedup vs baseline: 3.2035x; 1.8675x over previous
import functools

import numpy as np
import jax
import jax.numpy as jnp
from jax.experimental import pallas as pl
from jax.experimental.pallas import tpu as pltpu

_BN_EPS = 1e-5


def _stats_kernel(x_ref, o_ref):
    j = pl.program_id(1)

    @pl.when(j == 0)
    def _init():
        o_ref[...] = jnp.zeros_like(o_ref)

    x = x_ref[...]
    o_ref[0, 0] += jnp.sum(x, axis=0)
    o_ref[0, 1] += jnp.sum(x * x, axis=0)


def _tdot(a, b):
    return jax.lax.dot_general(a, b, (((0,), (0,)), ((), ())),
                               preferred_element_type=jnp.float32)


def _forward_kernel(x_ref, part_ref, gb_ref, wc_ref, w1_ref, w2_ref, b_ref,
                    o_ref, *, batch, ncores, groups, blk, chans, length,
                    filters, hidden, classes):
    s = part_ref[0, 0]
    q = part_ref[0, 1]
    for c in range(1, ncores):
        s = s + part_ref[c, 0]
        q = q + part_ref[c, 1]
    inv_b = 1.0 / batch
    mean = s * inv_b
    var = q * inv_b - mean * mean
    scale = gb_ref[0] * jax.lax.rsqrt(var + _BN_EPS)
    shift = gb_ref[1] - mean * scale

    xa = x_ref[...] * scale[None] + shift[None]
    tb = xa.shape[0]

    xt = jnp.stack([jnp.transpose(xa[:, c, :])
                    for c in range(chans)])

    hb = blk // 2
    xt = jnp.pad(xt, ((0, 0), (hb, hb), (0, 0)))
    x_sb = jnp.stack([xt[:, m * blk:(m + 1) * blk, :]
                      for m in range(groups + 1)])
    x_sb = x_sb.reshape((groups + 1) * chans * blk, tb)

    cb = jnp.tile(jnp.transpose(b_ref[0:1, :filters]),
                  (blk, 1))
    rows = 2 * chans * blk
    pooled_parts = []
    for g in range(groups):
        xw = x_sb[g * chans * blk:g * chans * blk + rows, :]
        cg = _tdot(wc_ref[...], xw)
        cg = jnp.maximum(cg + cb, 0.0)
        half = cg.shape[0] // 2
        pooled_parts.append(jnp.maximum(cg[:half], cg[half:]))
    pooled = jnp.concatenate(pooled_parts, axis=0)

    h = _tdot(w1_ref[...], pooled)
    h = jnp.maximum(h + jnp.transpose(b_ref[1:2, :hidden]), 0.0)
    logits = _tdot(w2_ref[...], h)
    logits = logits + jnp.transpose(b_ref[2:3, :classes])
    sig = jax.nn.sigmoid(logits)
    o_ref[...] = jnp.transpose(sig).astype(o_ref.dtype)


def _build_group_weight(w_toep, C, L, F, K, tg, pooling, blk):
    pool_len = L // pooling
    n_conv = w_toep.shape[1]
    w3 = w_toep.reshape(C, L, n_conv)[:, :K, pool_len * F:pool_len * F + F]
    P = np.zeros((K, 2, blk, tg), np.float32)
    for u in range(tg // pooling):
        for p in range(pooling):
            t_local = pooling * u + p
            j = p * (tg // pooling) + u
            for k in range(K):
                D = t_local + k - 1 + blk // 2
                P[k, D // blk, D % blk, j] = 1.0
    w = jnp.einsum('kmdj,ckf->mcdjf', jnp.asarray(P), w3)
    return w.reshape(2 * C * blk, tg * F)


def kernel(x, w_toep, fc1_w, fc2_w, bias_pack, gb_pack):
    B, C, L = x.shape
    H = fc1_w.shape[1]
    classes = fc2_w.shape[1]
    n_conv = w_toep.shape[1]
    pooling = 2
    pool_len = L // pooling
    F = n_conv // (pooling * pool_len)
    K = 3
    TG = 16
    G = L // TG
    BLK = 16

    wc = _build_group_weight(w_toep, C, L, F, K, TG, pooling, BLK)
    gb3 = gb_pack.reshape(2, C, L)

    TBS = 512 if B % 512 == 0 else (128 if B % 128 == 0 else B)
    ns_tiles = B // TBS
    ncores = 2 if ns_tiles % 2 == 0 else 1
    half_tiles = ns_tiles // ncores
    vmem_limit = 64 * 1024 * 1024

    parts = pl.pallas_call(
        _stats_kernel,
        out_shape=jax.ShapeDtypeStruct((ncores, 2, C, L), jnp.float32),
        grid_spec=pltpu.PrefetchScalarGridSpec(
            num_scalar_prefetch=0,
            grid=(ncores, half_tiles),
            in_specs=[pl.BlockSpec((TBS, C, L),
                                   lambda i, j: (i * half_tiles + j, 0, 0))],
            out_specs=pl.BlockSpec((1, 2, C, L), lambda i, j: (i, 0, 0, 0))),
        compiler_params=pltpu.CompilerParams(
            dimension_semantics=("parallel", "arbitrary"),
            vmem_limit_bytes=vmem_limit),
    )(x)

    TB = 256 if B % 256 == 0 else B
    n_tiles = B // TB
    fwd = functools.partial(_forward_kernel, batch=B, ncores=ncores, groups=G,
                            blk=BLK, chans=C, length=L, filters=F, hidden=H,
                            classes=classes)
    out = pl.pallas_call(
        fwd,
        out_shape=jax.ShapeDtypeStruct((B, classes), jnp.float32),
        grid_spec=pltpu.PrefetchScalarGridSpec(
            num_scalar_prefetch=0,
            grid=(n_tiles,),
            in_specs=[
                pl.BlockSpec((TB, C, L), lambda i: (i, 0, 0)),
                pl.BlockSpec((ncores, 2, C, L), lambda i: (0, 0, 0, 0)),
                pl.BlockSpec((2, C, L), lambda i: (0, 0, 0)),
                pl.BlockSpec((2 * C * BLK, TG * F), lambda i: (0, 0)),
                pl.BlockSpec((pool_len * F, H), lambda i: (0, 0)),
                pl.BlockSpec((H, classes), lambda i: (0, 0)),
                pl.BlockSpec(tuple(bias_pack.shape), lambda i: (0, 0)),
            ],
            out_specs=pl.BlockSpec((TB, classes), lambda i: (i, 0))),
        compiler_params=pltpu.CompilerParams(
            dimension_semantics=("parallel",),
            vmem_limit_bytes=vmem_limit),
    )(x, parts, gb3, wc, fc1_w, fc2_w, bias_pack)
    return out


# speedup vs baseline: 3.3594x; 1.0487x over previous
import functools

import numpy as np
import jax
import jax.numpy as jnp
from jax.experimental import pallas as pl
from jax.experimental.pallas import tpu as pltpu

_BN_EPS = 1e-5


def _stats_kernel(x_ref, o_ref):
    j = pl.program_id(1)

    @pl.when(j == 0)
    def _init():
        o_ref[...] = jnp.zeros_like(o_ref)

    x = x_ref[...]
    o_ref[0, 0] += jnp.sum(x, axis=0)
    o_ref[0, 1] += jnp.sum(x * x, axis=0)


def _tdot(a, b):
    return jax.lax.dot_general(a, b, (((0,), (0,)), ((), ())),
                               preferred_element_type=jnp.float32)


def _forward_kernel(x_ref, part_ref, gb_ref, wc_ref, w1_ref, w2_ref, b_ref,
                    o_ref, *, batch, ncores, groups, blk, chans, length,
                    filters, hidden, classes):
    s = part_ref[0, 0]
    q = part_ref[0, 1]
    for c in range(1, ncores):
        s = s + part_ref[c, 0]
        q = q + part_ref[c, 1]
    inv_b = 1.0 / batch
    mean = s * inv_b
    var = q * inv_b - mean * mean
    scale = gb_ref[0] * jax.lax.rsqrt(var + _BN_EPS)
    shift = gb_ref[1] - mean * scale

    xa = x_ref[...] * scale[None] + shift[None]
    tb = xa.shape[0]

    xt = jnp.stack([jnp.transpose(xa[:, c, :])
                    for c in range(chans)])

    hb = blk // 2
    xt = jnp.pad(xt, ((0, 0), (hb, hb), (0, 0)))
    x_sb = jnp.stack([xt[:, m * blk:(m + 1) * blk, :]
                      for m in range(groups + 1)])
    x_sb = x_sb.reshape((groups + 1) * chans * blk, tb)

    cb = jnp.tile(jnp.transpose(b_ref[0:1, :filters]),
                  (blk, 1))
    rows = 2 * chans * blk
    pooled_parts = []
    for g in range(groups):
        xw = x_sb[g * chans * blk:g * chans * blk + rows, :]
        cg = _tdot(wc_ref[...], xw)
        cg = jnp.maximum(cg + cb, 0.0)
        half = cg.shape[0] // 2
        pooled_parts.append(jnp.maximum(cg[:half], cg[half:]))
    pooled = jnp.concatenate(pooled_parts, axis=0)

    h = _tdot(w1_ref[...], pooled)
    h = jnp.maximum(h + jnp.transpose(b_ref[1:2, :hidden]), 0.0)
    logits = _tdot(w2_ref[...], h)
    logits = logits + jnp.transpose(b_ref[2:3, :classes])
    sig = jax.nn.sigmoid(logits)
    o_ref[...] = jnp.transpose(sig).astype(o_ref.dtype)


def _build_group_weight(w_toep, C, L, F, K, tg, pooling, blk):
    pool_len = L // pooling
    n_conv = w_toep.shape[1]
    w3 = w_toep.reshape(C, L, n_conv)[:, :K, pool_len * F:pool_len * F + F]
    P = np.zeros((K, 2, blk, tg), np.float32)
    for u in range(tg // pooling):
        for p in range(pooling):
            t_local = pooling * u + p
            j = p * (tg // pooling) + u
            for k in range(K):
                D = t_local + k - 1 + blk // 2
                P[k, D // blk, D % blk, j] = 1.0
    w = jnp.einsum('kmdj,ckf->mcdjf', jnp.asarray(P), w3)
    return w.reshape(2 * C * blk, tg * F)


def kernel(x, w_toep, fc1_w, fc2_w, bias_pack, gb_pack):
    B, C, L = x.shape
    H = fc1_w.shape[1]
    classes = fc2_w.shape[1]
    n_conv = w_toep.shape[1]
    pooling = 2
    pool_len = L // pooling
    F = n_conv // (pooling * pool_len)
    K = 3
    TG = 16
    G = L // TG
    BLK = 16

    wc = _build_group_weight(w_toep, C, L, F, K, TG, pooling, BLK)
    gb3 = gb_pack.reshape(2, C, L)

    TBS = 512 if B % 512 == 0 else (128 if B % 128 == 0 else B)
    ns_tiles = B // TBS
    ncores = 2 if ns_tiles % 2 == 0 else 1
    half_tiles = ns_tiles // ncores
    vmem_limit = 64 * 1024 * 1024

    parts = pl.pallas_call(
        _stats_kernel,
        out_shape=jax.ShapeDtypeStruct((ncores, 2, C, L), jnp.float32),
        grid_spec=pltpu.PrefetchScalarGridSpec(
            num_scalar_prefetch=0,
            grid=(ncores, half_tiles),
            in_specs=[pl.BlockSpec((TBS, C, L),
                                   lambda i, j: (i * half_tiles + j, 0, 0))],
            out_specs=pl.BlockSpec((1, 2, C, L), lambda i, j: (i, 0, 0, 0))),
        compiler_params=pltpu.CompilerParams(
            dimension_semantics=("parallel", "arbitrary"),
            vmem_limit_bytes=vmem_limit),
    )(x)

    TB = 512 if B % 512 == 0 else (256 if B % 256 == 0 else B)
    n_tiles = B // TB
    fwd = functools.partial(_forward_kernel, batch=B, ncores=ncores, groups=G,
                            blk=BLK, chans=C, length=L, filters=F, hidden=H,
                            classes=classes)
    out = pl.pallas_call(
        fwd,
        out_shape=jax.ShapeDtypeStruct((B, classes), jnp.float32),
        grid_spec=pltpu.PrefetchScalarGridSpec(
            num_scalar_prefetch=0,
            grid=(n_tiles,),
            in_specs=[
                pl.BlockSpec((TB, C, L), lambda i: (i, 0, 0)),
                pl.BlockSpec((ncores, 2, C, L), lambda i: (0, 0, 0, 0)),
                pl.BlockSpec((2, C, L), lambda i: (0, 0, 0)),
                pl.BlockSpec((2 * C * BLK, TG * F), lambda i: (0, 0)),
                pl.BlockSpec((pool_len * F, H), lambda i: (0, 0)),
                pl.BlockSpec((H, classes), lambda i: (0, 0)),
                pl.BlockSpec(tuple(bias_pack.shape), lambda i: (0, 0)),
            ],
            out_specs=pl.BlockSpec((TB, classes), lambda i: (i, 0))),
        compiler_params=pltpu.CompilerParams(
            dimension_semantics=("parallel",),
            vmem_limit_bytes=vmem_limit),
    )(x, parts, gb3, wc, fc1_w, fc2_w, bias_pack)
    return out
